# baseline (speedup 1.0000x reference)
import functools

import jax
import jax.numpy as jnp
from jax import lax
from jax.experimental import pallas as pl
from jax.experimental.pallas import tpu as pltpu
from jax.experimental.pallas import tpu_sc as plsc

_VOCAB_SIZES = [100000] * 26
_EMBED = 32
_B, _T, _H = 1024, 20, 26
_V = 100000

_PER_H = _B * _T
_IDX_ROWS = _PER_H // 128

_STRIDE = 2621440
_CCOL = 81920


def _copy_body(in_ref, out_ref):
    out_ref[...] = in_ref[...].reshape(8, _CCOL // 128, 128)


_recopy = pl.pallas_call(
    _copy_body,
    grid=(4, _STRIDE // _CCOL),
    in_specs=[pl.BlockSpec((8, _CCOL), lambda g, c: (g, c))],
    out_specs=pl.BlockSpec((8, _CCOL // 128, 128), lambda g, c: (g, c, 0)),
    out_shape=jax.ShapeDtypeStruct((32, _STRIDE // 128, 128), jnp.float32),
)

_mesh = plsc.VectorSubcoreMesh(core_axis_name="c", subcore_axis_name="s")


@functools.partial(
    pl.kernel,
    mesh=_mesh,
    out_type=jax.ShapeDtypeStruct((_T, _H, _EMBED, _B), jnp.float32),
    scratch_types=[
        pltpu.VMEM((_V,), jnp.float32),
        pltpu.VMEM((_IDX_ROWS, 128), jnp.int32),
        pltpu.VMEM((2, _B), jnp.float32),
        pltpu.SemaphoreType.DMA,
        pltpu.SemaphoreType.DMA,
        pltpu.SemaphoreType.DMA,
        pltpu.SemaphoreType.DMA,
    ],
    compiler_params=pltpu.CompilerParams(
        use_tc_tiling_on_sc=False, needs_layout_passes=False
    ),
)
def _sc_lookup(table_t_hbm, idx_hbm, out_hbm, strip_v, idx_v, ring_v,
               sem_in, sem_idx, sem_o0, sem_o1):
    d = lax.axis_index("s") * 2 + lax.axis_index("c")

    def per_head(h, carry):
        base = d * _STRIDE + h * _V
        chunk = _V // 10
        cps = [
            pltpu.async_copy(
                table_t_hbm.at[pl.ds(base + c * chunk, chunk)],
                strip_v.at[pl.ds(c * chunk, chunk)],
                sem_in,
            )
            for c in range(10)
        ]
        cpi = pltpu.async_copy(idx_hbm.at[h], idx_v, sem_idx)
        for cp in cps:
            cp.wait()
        cpi.wait()

        def per_t(t, carry_t):
            slot = lax.rem(t, 2)
            sem_o = [sem_o0, sem_o1]
            g = h * _T + t

            @pl.when(g >= 2)
            def _():
                for s, sem in enumerate(sem_o):
                    @pl.when(slot == s)
                    def _():
                        pltpu.make_async_copy(
                            ring_v.at[s], out_hbm.at[t, h, d], sem
                        ).wait()

            for v in range(_B // 16):
                iv = idx_v[t * 8 + v // 8, pl.ds((v % 8) * 16, 16)]
                ring_v[slot, pl.ds(v * 16, 16)] = plsc.load_gather(strip_v, [iv])

            for s, sem in enumerate(sem_o):
                @pl.when(slot == s)
                def _():
                    pltpu.async_copy(ring_v.at[s], out_hbm.at[t, h, d], sem)
            return carry_t

        lax.fori_loop(0, _T, per_t, 0)
        return carry

    lax.fori_loop(0, _H, per_head, 0)

    pltpu.make_async_copy(ring_v.at[0], out_hbm.at[0, 0, d], sem_o0).wait()
    pltpu.make_async_copy(ring_v.at[1], out_hbm.at[0, 0, d], sem_o1).wait()


def kernel(indices, table):
    table_t = table.T
    table_p = _recopy(table_t).reshape(-1)
    idx_t = jnp.transpose(indices, (2, 1, 0)).astype(jnp.int32)
    idx3 = idx_t.reshape(_H, _IDX_ROWS, 128)
    out = _sc_lookup(table_p, idx3)
    return jnp.transpose(out, (3, 0, 1, 2))


# speedup vs baseline: 1.0986x; 1.0986x over previous
import functools

import jax
import jax.numpy as jnp
from jax import lax
from jax.experimental import pallas as pl
from jax.experimental.pallas import tpu as pltpu
from jax.experimental.pallas import tpu_sc as plsc

_VOCAB_SIZES = [100000] * 26
_EMBED = 32
_B, _T, _H = 1024, 20, 26
_V = 100000

_PER_H = _B * _T
_IDX_ROWS = _PER_H // 128

_STRIDE = 2621440
_CCOL = 81920


def _copy_body(in_ref, out_ref):
    out_ref[...] = in_ref[...].reshape(8, _CCOL // 128, 128)


_recopy = pl.pallas_call(
    _copy_body,
    grid=(4, _STRIDE // _CCOL),
    in_specs=[pl.BlockSpec((8, _CCOL), lambda g, c: (g, c))],
    out_specs=pl.BlockSpec((8, _CCOL // 128, 128), lambda g, c: (g, c, 0)),
    out_shape=jax.ShapeDtypeStruct((32, _STRIDE // 128, 128), jnp.float32),
)

_mesh = plsc.VectorSubcoreMesh(core_axis_name="c", subcore_axis_name="s")


@functools.partial(
    pl.kernel,
    mesh=_mesh,
    out_type=jax.ShapeDtypeStruct((_T, _H, _EMBED // 8, _B // 128, 8, 128),
                                  jnp.float32),
    scratch_types=[
        pltpu.VMEM((_V,), jnp.float32),
        pltpu.VMEM((_IDX_ROWS, 128), jnp.int32),
        pltpu.VMEM((2, 8, 128), jnp.float32),
        pltpu.SemaphoreType.DMA,
        pltpu.SemaphoreType.DMA,
        pltpu.SemaphoreType.DMA,
        pltpu.SemaphoreType.DMA,
    ],
    compiler_params=pltpu.CompilerParams(
        use_tc_tiling_on_sc=False, needs_layout_passes=False
    ),
)
def _sc_lookup(table_t_hbm, idx_hbm, out_hbm, strip_v, idx_v, ring_v,
               sem_in, sem_idx, sem_o0, sem_o1):
    d = lax.axis_index("s") * 2 + lax.axis_index("c")

    def per_head(h, carry):
        base = d * _STRIDE + h * _V
        chunk = _V // 10
        cps = [
            pltpu.async_copy(
                table_t_hbm.at[pl.ds(base + c * chunk, chunk)],
                strip_v.at[pl.ds(c * chunk, chunk)],
                sem_in,
            )
            for c in range(10)
        ]
        cpi = pltpu.async_copy(idx_hbm.at[h], idx_v, sem_idx)
        for cp in cps:
            cp.wait()
        cpi.wait()

        def per_t(t, carry_t):
            slot = lax.rem(t, 2)
            sem_o = [sem_o0, sem_o1]
            g = h * _T + t

            @pl.when(g >= 2)
            def _():
                for s, sem in enumerate(sem_o):
                    @pl.when(slot == s)
                    def _():
                        pltpu.make_async_copy(
                            ring_v.at[s], out_hbm.at[t, h, d // 8, :, d % 8, :], sem
                        ).wait()

            for v in range(_B // 16):
                iv = idx_v[t * 8 + v // 8, pl.ds((v % 8) * 16, 16)]
                ring_v[slot, v // 8, pl.ds((v % 8) * 16, 16)] = plsc.load_gather(
                    strip_v, [iv]
                )

            for s, sem in enumerate(sem_o):
                @pl.when(slot == s)
                def _():
                    pltpu.async_copy(ring_v.at[s], out_hbm.at[t, h, d // 8, :, d % 8, :], sem)
            return carry_t

        lax.fori_loop(0, _T, per_t, 0)
        return carry

    lax.fori_loop(0, _H, per_head, 0)

    pltpu.make_async_copy(ring_v.at[0], out_hbm.at[0, 0, d // 8, :, d % 8, :], sem_o0).wait()
    pltpu.make_async_copy(ring_v.at[1], out_hbm.at[0, 0, d // 8, :, d % 8, :], sem_o1).wait()


def kernel(indices, table):
    table_t = table.T
    table_p = _recopy(table_t).reshape(-1)
    idx_t = jnp.transpose(indices, (2, 1, 0)).astype(jnp.int32)
    idx3 = idx_t.reshape(_H, _IDX_ROWS, 128)
    out6 = _sc_lookup(table_p, idx3)
    x = jnp.transpose(out6, (3, 5, 0, 1, 2, 4))
    return x.reshape(_B, _T, _H, _EMBED)


# speedup vs baseline: 1.6442x; 1.4966x over previous
import functools

import jax
import jax.numpy as jnp
from jax import lax
from jax.experimental import pallas as pl
from jax.experimental.pallas import tpu as pltpu
from jax.experimental.pallas import tpu_sc as plsc

_VOCAB_SIZES = [100000] * 26
_EMBED = 32
_B, _T, _H = 1024, 20, 26
_V = 100000

_PER_H = _B * _T
_IDX_ROWS = _PER_H // 128

_STRIDE = 2621440
_CCOL = 81920


def _copy_body(in_ref, out_ref):
    out_ref[...] = in_ref[...].reshape(8, _CCOL // 128, 128)


_recopy = pl.pallas_call(
    _copy_body,
    grid=(4, _STRIDE // _CCOL),
    in_specs=[pl.BlockSpec((8, _CCOL), lambda g, c: (g, c))],
    out_specs=pl.BlockSpec((8, _CCOL // 128, 128), lambda g, c: (g, c, 0)),
    out_shape=jax.ShapeDtypeStruct((32, _STRIDE // 128, 128), jnp.float32),
)

_mesh = plsc.VectorSubcoreMesh(core_axis_name="c", subcore_axis_name="s")


@functools.partial(
    pl.kernel,
    mesh=_mesh,
    out_type=jax.ShapeDtypeStruct((_T, _H, _EMBED // 8, _B // 128, 8, 128),
                                  jnp.float32),
    scratch_types=[
        pltpu.VMEM((_V,), jnp.float32),
        pltpu.VMEM((_IDX_ROWS, 128), jnp.int32),
        pltpu.VMEM((2, 8, 128), jnp.float32),
        pltpu.SemaphoreType.DMA,
        pltpu.SemaphoreType.DMA,
        pltpu.SemaphoreType.DMA,
        pltpu.SemaphoreType.DMA,
    ],
    compiler_params=pltpu.CompilerParams(
        use_tc_tiling_on_sc=False, needs_layout_passes=False
    ),
)
def _sc_lookup(table_t_hbm, idx_hbm, out_hbm, strip_v, idx_v, ring_v,
               sem_in, sem_idx, sem_o0, sem_o1):
    d = lax.axis_index("s") * 2 + lax.axis_index("c")

    def per_head(h, carry):
        base = d * _STRIDE + h * _V
        chunk = _V // 10
        cps = [
            pltpu.async_copy(
                table_t_hbm.at[pl.ds(base + c * chunk, chunk)],
                strip_v.at[pl.ds(c * chunk, chunk)],
                sem_in,
            )
            for c in range(10)
        ]
        cpi = pltpu.async_copy(idx_hbm.at[h], idx_v, sem_idx)
        for cp in cps:
            cp.wait()
        cpi.wait()

        def per_t(t, carry_t):
            slot = lax.rem(t, 2)
            sem_o = [sem_o0, sem_o1]
            g = h * _T + t

            @pl.when(g >= 2)
            def _():
                for s, sem in enumerate(sem_o):
                    @pl.when(slot == s)
                    def _():
                        pltpu.make_async_copy(
                            ring_v.at[s], out_hbm.at[t, h, d // 8, :, d % 8, :], sem
                        ).wait()

            @plsc.parallel_loop(0, _B // 16, unroll=8)
            def _gather(v):
                r = v // 8
                c = lax.rem(v, 8) * 16
                iv = idx_v[t * 8 + r, pl.ds(c, 16)]
                ring_v[slot, r, pl.ds(c, 16)] = plsc.load_gather(strip_v, [iv])

            for s, sem in enumerate(sem_o):
                @pl.when(slot == s)
                def _():
                    pltpu.async_copy(ring_v.at[s], out_hbm.at[t, h, d // 8, :, d % 8, :], sem)
            return carry_t

        lax.fori_loop(0, _T, per_t, 0)
        return carry

    lax.fori_loop(0, _H, per_head, 0)

    pltpu.make_async_copy(ring_v.at[0], out_hbm.at[0, 0, d // 8, :, d % 8, :], sem_o0).wait()
    pltpu.make_async_copy(ring_v.at[1], out_hbm.at[0, 0, d // 8, :, d % 8, :], sem_o1).wait()


def kernel(indices, table):
    table_t = table.T
    table_p = _recopy(table_t).reshape(-1)
    idx_t = jnp.transpose(indices, (2, 1, 0)).astype(jnp.int32)
    idx3 = idx_t.reshape(_H, _IDX_ROWS, 128)
    out6 = _sc_lookup(table_p, idx3)
    x = jnp.transpose(out6, (3, 5, 0, 1, 2, 4))
    return x.reshape(_B, _T, _H, _EMBED)


# speedup vs baseline: 1.6462x; 1.0012x over previous
import functools

import jax
import jax.numpy as jnp
from jax import lax
from jax.experimental import pallas as pl
from jax.experimental.pallas import tpu as pltpu
from jax.experimental.pallas import tpu_sc as plsc

_VOCAB_SIZES = [100000] * 26
_EMBED = 32
_B, _T, _H = 1024, 20, 26
_V = 100000

_PER_H = _B * _T
_IDX_ROWS = _PER_H // 128

_STRIDE = 2621440
_CCOL = 81920


def _copy_body(in_ref, out_ref):
    out_ref[...] = in_ref[...].reshape(8, _CCOL // 128, 128)


_recopy = pl.pallas_call(
    _copy_body,
    grid=(4, _STRIDE // _CCOL),
    in_specs=[pl.BlockSpec((8, _CCOL), lambda g, c: (g, c))],
    out_specs=pl.BlockSpec((8, _CCOL // 128, 128), lambda g, c: (g, c, 0)),
    out_shape=jax.ShapeDtypeStruct((32, _STRIDE // 128, 128), jnp.float32),
)

_mesh = plsc.VectorSubcoreMesh(core_axis_name="c", subcore_axis_name="s")


@functools.partial(
    pl.kernel,
    mesh=_mesh,
    out_type=jax.ShapeDtypeStruct((_T, _H, _EMBED // 8, _B // 128, 8, 128),
                                  jnp.float32),
    scratch_types=[
        pltpu.VMEM((_V,), jnp.float32),
        pltpu.VMEM((_IDX_ROWS, 128), jnp.int32),
        pltpu.VMEM((2, 8, 128), jnp.float32),
        pltpu.SemaphoreType.DMA,
        pltpu.SemaphoreType.DMA,
        pltpu.SemaphoreType.DMA,
        pltpu.SemaphoreType.DMA,
    ],
    compiler_params=pltpu.CompilerParams(
        use_tc_tiling_on_sc=False, needs_layout_passes=False
    ),
)
def _sc_lookup(table_t_hbm, idx_hbm, out_hbm, strip_v, idx_v, ring_v,
               sem_in, sem_idx, sem_o0, sem_o1):
    d = lax.axis_index("s") * 2 + lax.axis_index("c")

    def per_head(h, carry):
        base = d * _STRIDE + h * _V
        chunk = _V // 10
        cps = [
            pltpu.async_copy(
                table_t_hbm.at[pl.ds(base + c * chunk, chunk)],
                strip_v.at[pl.ds(c * chunk, chunk)],
                sem_in,
            )
            for c in range(10)
        ]
        cpi = pltpu.async_copy(idx_hbm.at[h], idx_v, sem_idx)
        for cp in cps:
            cp.wait()
        cpi.wait()

        def per_t(t, carry_t):
            slot = lax.rem(t, 2)
            sem_o = [sem_o0, sem_o1]
            g = h * _T + t

            @pl.when(g >= 2)
            def _():
                for s, sem in enumerate(sem_o):
                    @pl.when(slot == s)
                    def _():
                        pltpu.make_async_copy(
                            ring_v.at[s], out_hbm.at[t, h, d // 8, :, d % 8, :], sem
                        ).wait()

            @plsc.parallel_loop(0, _B // 16, unroll=16)
            def _gather(v):
                r = v // 8
                c = lax.rem(v, 8) * 16
                iv = idx_v[t * 8 + r, pl.ds(c, 16)]
                ring_v[slot, r, pl.ds(c, 16)] = plsc.load_gather(strip_v, [iv])

            for s, sem in enumerate(sem_o):
                @pl.when(slot == s)
                def _():
                    pltpu.async_copy(ring_v.at[s], out_hbm.at[t, h, d // 8, :, d % 8, :], sem)
            return carry_t

        lax.fori_loop(0, _T, per_t, 0)
        return carry

    lax.fori_loop(0, _H, per_head, 0)

    pltpu.make_async_copy(ring_v.at[0], out_hbm.at[0, 0, d // 8, :, d % 8, :], sem_o0).wait()
    pltpu.make_async_copy(ring_v.at[1], out_hbm.at[0, 0, d // 8, :, d % 8, :], sem_o1).wait()


def kernel(indices, table):
    table_t = table.T
    table_p = _recopy(table_t).reshape(-1)
    idx_t = jnp.transpose(indices, (2, 1, 0)).astype(jnp.int32)
    idx3 = idx_t.reshape(_H, _IDX_ROWS, 128)
    out6 = _sc_lookup(table_p, idx3)
    x = jnp.transpose(out6, (3, 5, 0, 1, 2, 4))
    return x.reshape(_B, _T, _H, _EMBED)
